# Initial kernel scaffold; baseline (speedup 1.0000x reference)
import jax
import jax.numpy as jnp
from jax.experimental import pallas as pl


def kernel(H, M, Wk, Wv):
    raise NotImplementedError("write your pallas kernel here")


# speedup vs baseline: 1.2018x; 1.2018x over previous
import functools

import jax
import jax.numpy as jnp
from jax.experimental import pallas as pl
from jax.experimental.pallas import tpu as pltpu


def _proj_kernel(m_ref, w_ref, kv_ref):
    acc = jax.lax.dot_general(
        m_ref[...], w_ref[...], (((1,), (0,)), ((), ())),
        preferred_element_type=jnp.float32)
    kv_ref[...] = acc.astype(jnp.bfloat16)


def _attn_kernel(num_kv, h_ref, k_ref, v_ref, o_ref, acc_ref, m_ref, l_ref):
    kv_i = pl.program_id(1)

    @pl.when(kv_i == 0)
    def _init():
        acc_ref[...] = jnp.zeros_like(acc_ref)
        m_ref[...] = jnp.full_like(m_ref, -jnp.inf)
        l_ref[...] = jnp.zeros_like(l_ref)

    q = h_ref[...].astype(jnp.bfloat16)
    s = jax.lax.dot_general(
        q, k_ref[...], (((1,), (1,)), ((), ())),
        preferred_element_type=jnp.float32)

    m_prev = m_ref[...]
    m_new = jnp.maximum(m_prev, jnp.max(s, axis=1, keepdims=True))
    corr = jnp.exp(m_prev - m_new)
    p = jnp.exp(s - m_new)
    l_ref[...] = l_ref[...] * corr + jnp.sum(p, axis=1, keepdims=True)
    m_ref[...] = m_new
    acc_ref[...] = acc_ref[...] * corr + jax.lax.dot_general(
        p.astype(jnp.bfloat16), v_ref[...], (((1,), (0,)), ((), ())),
        preferred_element_type=jnp.float32)

    @pl.when(kv_i == num_kv - 1)
    def _done():
        o_ref[...] = h_ref[...] + acc_ref[...] / l_ref[...]


def kernel(H, M, Wk, Wv):
    orig_shape = H.shape
    D = H.shape[-1]
    N = M.shape[0]
    Q = H.reshape(-1, D)
    NQ = Q.shape[0]

    Wcat = jnp.concatenate([Wk.T, Wv.T], axis=1).astype(jnp.bfloat16)
    Mb = M.astype(jnp.bfloat16)
    BM = min(2048, N)
    kv = pl.pallas_call(
        _proj_kernel,
        grid=(N // BM,),
        in_specs=[
            pl.BlockSpec((BM, D), lambda i: (i, 0)),
            pl.BlockSpec((D, 2 * D), lambda i: (0, 0)),
        ],
        out_specs=pl.BlockSpec((BM, 2 * D), lambda i: (i, 0)),
        out_shape=jax.ShapeDtypeStruct((N, 2 * D), jnp.bfloat16),
    )(Mb, Wcat)

    BQ = min(1024, NQ)
    BKV = min(1024, N)
    num_kv = N // BKV
    out = pl.pallas_call(
        functools.partial(_attn_kernel, num_kv),
        grid=(NQ // BQ, num_kv),
        in_specs=[
            pl.BlockSpec((BQ, D), lambda i, j: (i, 0)),
            pl.BlockSpec((BKV, D), lambda i, j: (j, 0)),
            pl.BlockSpec((BKV, D), lambda i, j: (j, 1)),
        ],
        out_specs=pl.BlockSpec((BQ, D), lambda i, j: (i, 0)),
        out_shape=jax.ShapeDtypeStruct((NQ, D), jnp.float32),
        scratch_shapes=[
            pltpu.VMEM((BQ, D), jnp.float32),
            pltpu.VMEM((BQ, 1), jnp.float32),
            pltpu.VMEM((BQ, 1), jnp.float32),
        ],
        compiler_params=pltpu.CompilerParams(
            dimension_semantics=("parallel", "arbitrary")),
    )(Q, kv, kv)
    return out.reshape(orig_shape)


# speedup vs baseline: 1.3598x; 1.1315x over previous
import functools

import jax
import jax.numpy as jnp
from jax.experimental import pallas as pl
from jax.experimental.pallas import tpu as pltpu


def _proj_kernel(m_ref, w_ref, kv_ref):
    acc = jax.lax.dot_general(
        m_ref[...], w_ref[...], (((1,), (0,)), ((), ())),
        preferred_element_type=jnp.float32)
    kv_ref[...] = acc.astype(jnp.bfloat16)


def _attn_kernel(num_kv, h_ref, k_ref, v_ref, o_ref, acc_ref, m_ref, l_ref,
                 q_ref):
    kv_i = pl.program_id(1)

    @pl.when(kv_i == 0)
    def _init():
        acc_ref[...] = jnp.zeros_like(acc_ref)
        m_ref[...] = jnp.full_like(m_ref, -jnp.inf)
        l_ref[...] = jnp.zeros_like(l_ref)
        q_ref[...] = h_ref[...].astype(jnp.bfloat16)

    q = q_ref[...]
    s = jax.lax.dot_general(
        q, k_ref[...], (((1,), (1,)), ((), ())),
        preferred_element_type=jnp.float32)

    m_prev = m_ref[...]
    m_new = jnp.maximum(m_prev, jnp.max(s, axis=1, keepdims=True))
    corr = jnp.exp(m_prev - m_new)
    p = jnp.exp(s - m_new)
    l_ref[...] = l_ref[...] * corr + jnp.sum(p, axis=1, keepdims=True)
    m_ref[...] = m_new
    acc_ref[...] = acc_ref[...] * corr + jax.lax.dot_general(
        p.astype(jnp.bfloat16), v_ref[...], (((1,), (0,)), ((), ())),
        preferred_element_type=jnp.float32)

    @pl.when(kv_i == num_kv - 1)
    def _done():
        o_ref[...] = h_ref[...] + acc_ref[...] / l_ref[...]


def kernel(H, M, Wk, Wv):
    orig_shape = H.shape
    D = H.shape[-1]
    N = M.shape[0]
    Q = H.reshape(-1, D)
    NQ = Q.shape[0]

    Wcat = jnp.concatenate([Wk.T, Wv.T], axis=1).astype(jnp.bfloat16)
    Mb = M.astype(jnp.bfloat16)
    BM = min(2048, N)
    kv = pl.pallas_call(
        _proj_kernel,
        grid=(N // BM,),
        in_specs=[
            pl.BlockSpec((BM, D), lambda i: (i, 0)),
            pl.BlockSpec((D, 2 * D), lambda i: (0, 0)),
        ],
        out_specs=pl.BlockSpec((BM, 2 * D), lambda i: (i, 0)),
        out_shape=jax.ShapeDtypeStruct((N, 2 * D), jnp.bfloat16),
    )(Mb, Wcat)

    BQ = min(1024, NQ)
    BKV = min(2048, N)
    num_kv = N // BKV
    out = pl.pallas_call(
        functools.partial(_attn_kernel, num_kv),
        grid=(NQ // BQ, num_kv),
        in_specs=[
            pl.BlockSpec((BQ, D), lambda i, j: (i, 0)),
            pl.BlockSpec((BKV, D), lambda i, j: (j, 0)),
            pl.BlockSpec((BKV, D), lambda i, j: (j, 1)),
        ],
        out_specs=pl.BlockSpec((BQ, D), lambda i, j: (i, 0)),
        out_shape=jax.ShapeDtypeStruct((NQ, D), jnp.float32),
        scratch_shapes=[
            pltpu.VMEM((BQ, D), jnp.float32),
            pltpu.VMEM((BQ, 1), jnp.float32),
            pltpu.VMEM((BQ, 1), jnp.float32),
            pltpu.VMEM((BQ, D), jnp.bfloat16),
        ],
        compiler_params=pltpu.CompilerParams(
            dimension_semantics=("parallel", "arbitrary")),
    )(Q, kv, kv)
    return out.reshape(orig_shape)
